# Initial kernel scaffold; baseline (speedup 1.0000x reference)
import jax
import jax.numpy as jnp
from jax.experimental import pallas as pl


def kernel(x, x_len, emb_weight):
    raise NotImplementedError("write your pallas kernel here")


# speedup vs baseline: 4.1497x; 4.1497x over previous
import functools

import jax
import jax.numpy as jnp
from jax import lax
from jax.experimental import pallas as pl
from jax.experimental.pallas import tpu as pltpu
from jax.experimental.pallas import tpu_sc as plsc

BATCH = 4096
SEQ = 50
DIM = 64
LANES = 16

NUM_CORES = 2
NUM_SUBCORES = 16
NW = NUM_CORES * NUM_SUBCORES
SEQ_PER_W = BATCH // NW
CHUNK_SEQ = 8
ROWS_PER_CHUNK = CHUNK_SEQ * SEQ
SUB = 4
ROWS_PER_SUB = ROWS_PER_CHUNK // SUB
N_CHUNKS = SEQ_PER_W // CHUNK_SEQ
XROWS = BATCH * SEQ // ROWS_PER_SUB


def _sc_body(x_hbm, len_hbm, tab_hbm, ret_hbm, out_hbm,
             idx_v, rows_v, ret_v, len_v, recip_v, sem):
    wid = lax.axis_index("s") * NUM_CORES + lax.axis_index("c")
    seq_base = wid * SEQ_PER_W
    xrow_base = wid * (SEQ_PER_W * SEQ // ROWS_PER_SUB)

    pltpu.sync_copy(len_hbm.at[pl.ds(seq_base, SEQ_PER_W)], len_v)
    for i in range(SEQ_PER_W // LANES):
        sl = pl.ds(i * LANES, LANES)
        recip_v[i, :] = 1.0 / len_v[sl].astype(jnp.float32)

    def chunk_body(c, carry):
        xrow = xrow_base + c * SUB
        pltpu.sync_copy(x_hbm.at[pl.ds(xrow, SUB)], idx_v)
        cps = [pltpu.async_copy(tab_hbm.at[idx_v.at[j]], rows_v.at[j], sem)
               for j in range(SUB)]
        for cp in cps:
            cp.wait()

        for s2 in range(CHUNK_SEQ):
            j = (s2 * SEQ) // ROWS_PER_SUB
            off = (s2 * SEQ) % ROWS_PER_SUB

            def tbody(t, accs):
                return tuple(
                    accs[k] + rows_v[j, off + t, pl.ds(k * LANES, LANES)]
                    for k in range(DIM // LANES))

            zero = jnp.zeros((LANES,), jnp.float32)
            accs = lax.fori_loop(0, SEQ, tbody, (zero,) * (DIM // LANES))
            blk = (c * CHUNK_SEQ + s2) // LANES
            lane = (c * CHUNK_SEQ + s2) % LANES
            vec16 = recip_v[blk, :]
            sel = jnp.where(lax.iota(jnp.int32, 16) == lane, vec16, 0.0)
            rv = jnp.full((LANES,), jnp.sum(sel))
            for k in range(DIM // LANES):
                ret_v[s2, pl.ds(k * LANES, LANES)] = accs[k] * rv

        pltpu.sync_copy(rows_v, out_hbm.at[pl.ds(xrow, SUB)])
        pltpu.sync_copy(ret_v, ret_hbm.at[pl.ds(seq_base + c * CHUNK_SEQ,
                                                CHUNK_SEQ)])
        return carry

    lax.fori_loop(0, N_CHUNKS, chunk_body, 0)


@jax.jit
def _run(x2, x_len, emb_weight):
    mesh = plsc.VectorSubcoreMesh(core_axis_name="c", subcore_axis_name="s")
    k = pl.kernel(
        _sc_body,
        mesh=mesh,
        compiler_params=pltpu.CompilerParams(
            needs_layout_passes=False, use_tc_tiling_on_sc=False),
        out_type=(
            jax.ShapeDtypeStruct((BATCH, DIM), jnp.float32),
            jax.ShapeDtypeStruct((XROWS, ROWS_PER_SUB, DIM), jnp.float32),
        ),
        scratch_types=[
            pltpu.VMEM((SUB, ROWS_PER_SUB), jnp.int32),
            pltpu.VMEM((SUB, ROWS_PER_SUB, DIM), jnp.float32),
            pltpu.VMEM((CHUNK_SEQ, DIM), jnp.float32),
            pltpu.VMEM((SEQ_PER_W,), jnp.int32),
            pltpu.VMEM((SEQ_PER_W // LANES, LANES), jnp.float32),
            pltpu.SemaphoreType.DMA,
        ],
    )
    return k(x2, x_len, emb_weight)


def kernel(x, x_len, emb_weight):
    x2 = x.astype(jnp.int32).reshape(XROWS, ROWS_PER_SUB)
    ret, out3 = _run(x2, x_len.astype(jnp.int32), emb_weight)
    return (ret, out3.reshape(BATCH, SEQ, DIM))


# speedup vs baseline: 4.6913x; 1.1305x over previous
import functools

import jax
import jax.numpy as jnp
from jax import lax
from jax.experimental import pallas as pl
from jax.experimental.pallas import tpu as pltpu
from jax.experimental.pallas import tpu_sc as plsc

BATCH = 4096
SEQ = 50
DIM = 64
LANES = 16

NUM_CORES = 2
NUM_SUBCORES = 16
NW = NUM_CORES * NUM_SUBCORES
SEQ_PER_W = BATCH // NW
CHUNK_SEQ = 8
ROWS_PER_CHUNK = CHUNK_SEQ * SEQ
SUB = 4
ROWS_PER_SUB = ROWS_PER_CHUNK // SUB
N_CHUNKS = SEQ_PER_W // CHUNK_SEQ
XROWS = BATCH * SEQ // ROWS_PER_SUB
NBUF = 3


def _sc_body(x_hbm, len_hbm, tab_hbm, ret_hbm, out_hbm,
             idx_v, rows_v, ret_v, len_v, recip_v, sem_g, sem_o):
    wid = lax.axis_index("s") * NUM_CORES + lax.axis_index("c")
    seq_base = wid * SEQ_PER_W
    xrow_base = wid * (SEQ_PER_W * SEQ // ROWS_PER_SUB)

    pltpu.sync_copy(len_hbm.at[pl.ds(seq_base, SEQ_PER_W)], len_v)
    for i in range(SEQ_PER_W // LANES):
        recip_v[i, :] = 1.0 / len_v[pl.ds(i * LANES, LANES)].astype(jnp.float32)

    def issue(c):
        b = c % NBUF
        xrow = xrow_base + c * SUB
        pltpu.sync_copy(x_hbm.at[pl.ds(xrow, SUB)], idx_v.at[b])
        cps = [pltpu.async_copy(tab_hbm.at[idx_v.at[b].at[j]],
                                rows_v.at[b].at[j], sem_g[b])
               for j in range(SUB)]
        return cps

    def drain_gather(cps):
        for cp in cps:
            cp.wait()

    def process(c):
        b = c % NBUF
        for s2 in range(CHUNK_SEQ):
            j = (s2 * SEQ) // ROWS_PER_SUB
            off = (s2 * SEQ) % ROWS_PER_SUB

            def tbody(t, accs, j=j, off=off, b=b):
                r = off + t * 2
                a = tuple(
                    accs[k] + rows_v[b, j, r, pl.ds(k * LANES, LANES)]
                    for k in range(DIM // LANES))
                return tuple(
                    a[k] + rows_v[b, j, r + 1, pl.ds(k * LANES, LANES)]
                    for k in range(DIM // LANES))

            zero = jnp.zeros((LANES,), jnp.float32)
            accs = lax.fori_loop(0, SEQ // 2, tbody, (zero,) * (DIM // LANES))
            blk = (c * CHUNK_SEQ + s2) // LANES
            lane = (c * CHUNK_SEQ + s2) % LANES
            sel = jnp.where(lax.iota(jnp.int32, 16) == lane,
                            recip_v[blk, :], 0.0)
            rv = jnp.full((LANES,), jnp.sum(sel))
            for k in range(DIM // LANES):
                ret_v[s2, pl.ds(k * LANES, LANES)] = accs[k] * rv
        pltpu.sync_copy(ret_v, ret_hbm.at[pl.ds(seq_base + c * CHUNK_SEQ,
                                                CHUNK_SEQ)])
        xrow = xrow_base + c * SUB
        return pltpu.async_copy(rows_v.at[b], out_hbm.at[pl.ds(xrow, SUB)],
                                sem_o[b])

    gcps = [None] * N_CHUNKS
    ocps = [None] * N_CHUNKS
    gcps[0] = issue(0)
    for c in range(1, N_CHUNKS + 1):
        if c < N_CHUNKS:
            if c >= NBUF:
                ocps[c - NBUF].wait()
            gcps[c] = issue(c)
        drain_gather(gcps[c - 1])
        ocps[c - 1] = process(c - 1)
    for c in range(N_CHUNKS - NBUF, N_CHUNKS):
        ocps[c].wait()


@jax.jit
def _run(x2, x_len, emb_weight):
    mesh = plsc.VectorSubcoreMesh(core_axis_name="c", subcore_axis_name="s")
    k = pl.kernel(
        _sc_body,
        mesh=mesh,
        compiler_params=pltpu.CompilerParams(
            needs_layout_passes=False, use_tc_tiling_on_sc=False),
        out_type=(
            jax.ShapeDtypeStruct((BATCH, DIM), jnp.float32),
            jax.ShapeDtypeStruct((XROWS, ROWS_PER_SUB, DIM), jnp.float32),
        ),
        scratch_types=[
            pltpu.VMEM((NBUF, SUB, ROWS_PER_SUB), jnp.int32),
            pltpu.VMEM((NBUF, SUB, ROWS_PER_SUB, DIM), jnp.float32),
            pltpu.VMEM((CHUNK_SEQ, DIM), jnp.float32),
            pltpu.VMEM((SEQ_PER_W,), jnp.int32),
            pltpu.VMEM((SEQ_PER_W // LANES, LANES), jnp.float32),
            [pltpu.SemaphoreType.DMA] * NBUF,
            [pltpu.SemaphoreType.DMA] * NBUF,
        ],
    )
    return k(x2, x_len, emb_weight)


def kernel(x, x_len, emb_weight):
    x2 = x.astype(jnp.int32).reshape(XROWS, ROWS_PER_SUB)
    ret, out3 = _run(x2, x_len.astype(jnp.int32), emb_weight)
    return (ret, out3.reshape(BATCH, SEQ, DIM))
